# Initial kernel scaffold; baseline (speedup 1.0000x reference)
import jax
import jax.numpy as jnp
from jax.experimental import pallas as pl


def kernel(i_node, edge_index):
    raise NotImplementedError("write your pallas kernel here")


# speedup vs baseline: 7.7967x; 7.7967x over previous
import functools

import jax
import jax.numpy as jnp
from jax import lax
from jax.experimental import pallas as pl
from jax.experimental.pallas import tpu as pltpu
from jax.experimental.pallas import tpu_sc as plsc

N_I = 10000
N_D = 10000
E = 320000
D = 128

NC = 2
NS = 16
NW = NC * NS
EPW = E // NW
C = 80
NCH = EPW // C
AR = 10240
RPT = AR // NS
ZR = 128


def _sc_body(src_hbm, dst_hbm, table_hbm, out_hbm,
             sidx, didx, rows, acc, gsem):
    c = lax.axis_index("c")
    s = lax.axis_index("s")
    w = c * NS + s

    pltpu.sync_copy(src_hbm.at[w], sidx)
    pltpu.sync_copy(dst_hbm.at[w], didx)

    zero = jnp.zeros((16,), jnp.float32)

    @pl.loop(0, C)
    def _(i):
        @pl.loop(0, D // 16)
        def _(k):
            rows[i, pl.ds(k * 16, 16)] = zero

    for r in range(RPT // C):
        pltpu.sync_copy(rows, acc.at[pl.ds(s * RPT + r * C, C)])
    plsc.subcore_barrier()

    @pl.loop(0, NCH)
    def _(j):
        pltpu.async_copy(table_hbm.at[sidx.at[j]], rows, gsem).wait()
        pltpu.sync_copy(rows, acc.at[didx.at[j]], add=True)

    plsc.subcore_barrier()

    for r in range(RPT // ZR):
        base = s * RPT + r * ZR
        pltpu.sync_copy(acc.at[pl.ds(base, ZR)],
                        out_hbm.at[c].at[pl.ds(base, ZR)])


@functools.cache
def _sc_call():
    return pl.kernel(
        _sc_body,
        out_type=jax.ShapeDtypeStruct((NC, AR, D), jnp.float32),
        mesh=plsc.VectorSubcoreMesh(core_axis_name="c", subcore_axis_name="s",
                                    num_cores=NC, num_subcores=NS),
        scratch_types=[
            pltpu.VMEM((NCH, C), jnp.int32),
            pltpu.VMEM((NCH, C), jnp.int32),
            pltpu.VMEM((C, D), jnp.float32),
            pltpu.VMEM_SHARED((AR, D), jnp.float32),
            pltpu.SemaphoreType.DMA,
        ],
    )


def _combine_body(p_ref, o_ref):
    o_ref[...] = p_ref[0] + p_ref[1]


def kernel(i_node, edge_index):
    src = edge_index[0].astype(jnp.int32).reshape(NW, NCH, C)
    dst = edge_index[1].astype(jnp.int32).reshape(NW, NCH, C)
    partials = _sc_call()(src, dst, i_node)
    nb = 10
    rb = N_D // nb
    return pl.pallas_call(
        _combine_body,
        out_shape=jax.ShapeDtypeStruct((N_D, D), jnp.float32),
        grid=(nb,),
        in_specs=[pl.BlockSpec((NC, rb, D), lambda i: (0, i, 0))],
        out_specs=pl.BlockSpec((rb, D), lambda i: (i, 0)),
    )(partials)


# speedup vs baseline: 10.2491x; 1.3145x over previous
import functools

import jax
import jax.numpy as jnp
from jax import lax
from jax.experimental import pallas as pl
from jax.experimental.pallas import tpu as pltpu
from jax.experimental.pallas import tpu_sc as plsc

N_I = 10000
N_D = 10000
E = 320000
D = 128

NC = 2
NS = 16
NW = NC * NS
EPW = E // NW
C = 100
NBLK = 5
IB = 20
AR = 10240
RPT = AR // NS
ZB = 80
ZR = 128


def _sc_body(src_hbm, dst_hbm, table_hbm, out_hbm,
             sidx, didx, rows0, rows1, acc, gsem):
    c = lax.axis_index("c")
    s = lax.axis_index("s")
    w = c * NS + s

    zero = jnp.zeros((16,), jnp.float32)

    @pl.loop(0, ZB)
    def _(i):
        @pl.loop(0, D // 16)
        def _(k):
            rows0[i, pl.ds(k * 16, 16)] = zero

    zsrc = rows0.at[pl.ds(0, ZB)]
    for r in range(RPT // ZB):
        pltpu.sync_copy(zsrc, acc.at[pl.ds(s * RPT + r * ZB, ZB)])
    plsc.subcore_barrier()

    for b in range(NBLK):
        pltpu.sync_copy(src_hbm.at[w, b], sidx)
        pltpu.sync_copy(dst_hbm.at[w, b], didx)
        pltpu.async_copy(table_hbm.at[sidx.at[0]], rows0, gsem).wait()

        @pl.loop(0, IB - 2, step=2)
        def _(g):
            cp = pltpu.async_copy(table_hbm.at[sidx.at[g + 1]], rows1, gsem)
            pltpu.sync_copy(rows0, acc.at[didx.at[g]], add=True)
            cp.wait()
            cp = pltpu.async_copy(table_hbm.at[sidx.at[g + 2]], rows0, gsem)
            pltpu.sync_copy(rows1, acc.at[didx.at[g + 1]], add=True)
            cp.wait()

        cp = pltpu.async_copy(table_hbm.at[sidx.at[IB - 1]], rows1, gsem)
        pltpu.sync_copy(rows0, acc.at[didx.at[IB - 2]], add=True)
        cp.wait()
        pltpu.sync_copy(rows1, acc.at[didx.at[IB - 1]], add=True)

    plsc.subcore_barrier()

    for r in range(RPT // ZR):
        base = s * RPT + r * ZR
        pltpu.sync_copy(acc.at[pl.ds(base, ZR)],
                        out_hbm.at[c].at[pl.ds(base, ZR)])


@functools.cache
def _sc_call():
    return pl.kernel(
        _sc_body,
        out_type=jax.ShapeDtypeStruct((NC, AR, D), jnp.float32),
        mesh=plsc.VectorSubcoreMesh(core_axis_name="c", subcore_axis_name="s",
                                    num_cores=NC, num_subcores=NS),
        scratch_types=[
            pltpu.VMEM((IB, C), jnp.int32),
            pltpu.VMEM((IB, C), jnp.int32),
            pltpu.VMEM((C, D), jnp.float32),
            pltpu.VMEM((C, D), jnp.float32),
            pltpu.VMEM_SHARED((AR, D), jnp.float32),
            pltpu.SemaphoreType.DMA,
        ],
    )


def _combine_body(p_ref, o_ref):
    o_ref[...] = p_ref[0] + p_ref[1]


def kernel(i_node, edge_index):
    src = edge_index[0].astype(jnp.int32).reshape(NW, NBLK, IB, C)
    dst = edge_index[1].astype(jnp.int32).reshape(NW, NBLK, IB, C)
    partials = _sc_call()(src, dst, i_node)
    nb = 10
    rb = N_D // nb
    return pl.pallas_call(
        _combine_body,
        out_shape=jax.ShapeDtypeStruct((N_D, D), jnp.float32),
        grid=(nb,),
        in_specs=[pl.BlockSpec((NC, rb, D), lambda i: (0, i, 0))],
        out_specs=pl.BlockSpec((rb, D), lambda i: (i, 0)),
    )(partials)


# speedup vs baseline: 11.9501x; 1.1660x over previous
import functools

import jax
import jax.numpy as jnp
from jax import lax
from jax.experimental import pallas as pl
from jax.experimental.pallas import tpu as pltpu
from jax.experimental.pallas import tpu_sc as plsc

N_I = 10000
N_D = 10000
E = 320000
D = 128

NC = 2
NS = 16
NW = NC * NS
EPW = E // NW
C = 50
NBLK = 5
IB = 40
AR = 10240
RPT = AR // NS
ZB = 40
ZR = 128
NBUF = 4


def _sc_body(src_hbm, dst_hbm, table_hbm, out_hbm,
             sidx, didx, r0, r1, r2, r3, acc, s0, s1, s2, s3):
    c = lax.axis_index("c")
    s = lax.axis_index("s")
    w = c * NS + s
    rows = (r0, r1, r2, r3)
    sems = (s0, s1, s2, s3)

    zero = jnp.zeros((16,), jnp.float32)

    @pl.loop(0, ZB)
    def _(i):
        @pl.loop(0, D // 16)
        def _(k):
            r0[i, pl.ds(k * 16, 16)] = zero

    zsrc = r0.at[pl.ds(0, ZB)]
    for r in range(RPT // ZB):
        pltpu.async_copy(zsrc, acc.at[pl.ds(s * RPT + r * ZB, ZB)], s0)
    for r in range(RPT // ZB):
        pltpu.make_async_copy(zsrc, acc.at[pl.ds(s * RPT, ZB)], s0).wait()
    plsc.subcore_barrier()

    def start_g(t, p):
        pltpu.async_copy(table_hbm.at[sidx.at[t]], rows[p], sems[p])

    def wait_g(t, p):
        pltpu.make_async_copy(table_hbm.at[sidx.at[t]], rows[p], sems[p]).wait()

    def start_s(t, p):
        pltpu.async_copy(rows[p], acc.at[didx.at[t]], sems[p], add=True)

    def wait_s(t, p):
        pltpu.make_async_copy(rows[p], acc.at[didx.at[t]], sems[p]).wait()

    for b in range(NBLK):
        pltpu.sync_copy(src_hbm.at[w, b], sidx)
        pltpu.sync_copy(dst_hbm.at[w, b], didx)

        start_g(0, 0)
        start_g(1, 1)
        wait_g(0, 0); start_s(0, 0); start_g(2, 2)
        wait_g(1, 1); start_s(1, 1); start_g(3, 3)
        wait_s(0, 0); start_g(4, 0); wait_g(2, 2); start_s(2, 2)
        wait_s(1, 1); start_g(5, 1); wait_g(3, 3); start_s(3, 3)

        @pl.loop(4, IB - 4, step=NBUF)
        def _(g):
            for p in range(NBUF):
                t = g + p
                q = (p + 2) % NBUF
                wait_s(t - 2, q)
                start_g(t + 2, q)
                wait_g(t, p)
                start_s(t, p)

        wait_s(IB - 6, 2); start_g(IB - 2, 2); wait_g(IB - 4, 0); start_s(IB - 4, 0)
        wait_s(IB - 5, 3); start_g(IB - 1, 3); wait_g(IB - 3, 1); start_s(IB - 3, 1)
        wait_s(IB - 4, 0); wait_g(IB - 2, 2); start_s(IB - 2, 2)
        wait_s(IB - 3, 1); wait_g(IB - 1, 3); start_s(IB - 1, 3)
        wait_s(IB - 2, 2)
        wait_s(IB - 1, 3)

    plsc.subcore_barrier()

    for r in range(RPT // ZR):
        base = s * RPT + r * ZR
        pltpu.async_copy(acc.at[pl.ds(base, ZR)],
                         out_hbm.at[c].at[pl.ds(base, ZR)], sems[r % NBUF])
    for r in range(RPT // ZR):
        base = s * RPT + r * ZR
        pltpu.make_async_copy(acc.at[pl.ds(base, ZR)],
                              out_hbm.at[c].at[pl.ds(base, ZR)],
                              sems[r % NBUF]).wait()


@functools.cache
def _sc_call():
    return pl.kernel(
        _sc_body,
        out_type=jax.ShapeDtypeStruct((NC, AR, D), jnp.float32),
        mesh=plsc.VectorSubcoreMesh(core_axis_name="c", subcore_axis_name="s",
                                    num_cores=NC, num_subcores=NS),
        scratch_types=[
            pltpu.VMEM((IB, C), jnp.int32),
            pltpu.VMEM((IB, C), jnp.int32),
            pltpu.VMEM((C, D), jnp.float32),
            pltpu.VMEM((C, D), jnp.float32),
            pltpu.VMEM((C, D), jnp.float32),
            pltpu.VMEM((C, D), jnp.float32),
            pltpu.VMEM_SHARED((AR, D), jnp.float32),
            pltpu.SemaphoreType.DMA,
            pltpu.SemaphoreType.DMA,
            pltpu.SemaphoreType.DMA,
            pltpu.SemaphoreType.DMA,
        ],
    )


def _combine_body(p_ref, o_ref):
    o_ref[...] = p_ref[0] + p_ref[1]


def kernel(i_node, edge_index):
    src = edge_index[0].astype(jnp.int32).reshape(NW, NBLK, IB, C)
    dst = edge_index[1].astype(jnp.int32).reshape(NW, NBLK, IB, C)
    partials = _sc_call()(src, dst, i_node)
    nb = 10
    rb = N_D // nb
    return pl.pallas_call(
        _combine_body,
        out_shape=jax.ShapeDtypeStruct((N_D, D), jnp.float32),
        grid=(nb,),
        in_specs=[pl.BlockSpec((NC, rb, D), lambda i: (0, i, 0))],
        out_specs=pl.BlockSpec((rb, D), lambda i: (i, 0)),
    )(partials)


# speedup vs baseline: 12.2700x; 1.0268x over previous
import functools

import jax
import jax.numpy as jnp
from jax import lax
from jax.experimental import pallas as pl
from jax.experimental.pallas import tpu as pltpu
from jax.experimental.pallas import tpu_sc as plsc

N_I = 10000
N_D = 10000
E = 320000
D = 128

NC = 2
NS = 16
NW = NC * NS
EPW = E // NW
C = 50
NBLK = 5
IB = 40
AR = 10240
RPT = AR // NS
ZB = 40
ZR = 128
NBUF = 4


def _sc_body(src_hbm, dst_hbm, table_hbm, out_hbm,
             sidx, didx, r0, r1, r2, r3, acc, s0, s1, s2, s3, isem):
    c = lax.axis_index("c")
    s = lax.axis_index("s")
    w = c * NS + s
    rows = (r0, r1, r2, r3)
    sems = (s0, s1, s2, s3)

    zero = jnp.zeros((16,), jnp.float32)

    @pl.loop(0, ZB)
    def _(i):
        @pl.loop(0, D // 16)
        def _(k):
            r0[i, pl.ds(k * 16, 16)] = zero

    zsrc = r0.at[pl.ds(0, ZB)]
    for r in range(RPT // ZB):
        pltpu.async_copy(zsrc, acc.at[pl.ds(s * RPT + r * ZB, ZB)], s0)
    for r in range(RPT // ZB):
        pltpu.make_async_copy(zsrc, acc.at[pl.ds(s * RPT, ZB)], s0).wait()
    plsc.subcore_barrier()

    pltpu.sync_copy(src_hbm.at[w, 0], sidx.at[0])
    pltpu.sync_copy(dst_hbm.at[w, 0], didx.at[0])
    for b in range(NBLK):
        q = b % 2
        sq, dq = sidx.at[q], didx.at[q]

        def start_g(t, p, sq=sq):
            pltpu.async_copy(table_hbm.at[sq.at[t]], rows[p], sems[p])

        def wait_g(t, p, sq=sq):
            pltpu.make_async_copy(table_hbm.at[sq.at[t]], rows[p],
                                  sems[p]).wait()

        def start_s(t, p, dq=dq):
            pltpu.async_copy(rows[p], acc.at[dq.at[t]], sems[p], add=True)

        def wait_s(t, p, dq=dq):
            pltpu.make_async_copy(rows[p], acc.at[dq.at[t]], sems[p]).wait()

        if b + 1 < NBLK:
            pltpu.async_copy(src_hbm.at[w, b + 1], sidx.at[1 - q], isem)
            pltpu.async_copy(dst_hbm.at[w, b + 1], didx.at[1 - q], isem)

        start_g(0, 0)
        start_g(1, 1)
        wait_g(0, 0); start_s(0, 0); start_g(2, 2)
        wait_g(1, 1); start_s(1, 1); start_g(3, 3)
        wait_s(0, 0); start_g(4, 0); wait_g(2, 2); start_s(2, 2)
        wait_s(1, 1); start_g(5, 1); wait_g(3, 3); start_s(3, 3)

        @pl.loop(4, IB - 4, step=NBUF)
        def _(g):
            for p in range(NBUF):
                t = g + p
                q = (p + 2) % NBUF
                wait_s(t - 2, q)
                start_g(t + 2, q)
                wait_g(t, p)
                start_s(t, p)

        wait_s(IB - 6, 2); start_g(IB - 2, 2); wait_g(IB - 4, 0); start_s(IB - 4, 0)
        wait_s(IB - 5, 3); start_g(IB - 1, 3); wait_g(IB - 3, 1); start_s(IB - 3, 1)
        wait_s(IB - 4, 0); wait_g(IB - 2, 2); start_s(IB - 2, 2)
        wait_s(IB - 3, 1); wait_g(IB - 1, 3); start_s(IB - 1, 3)
        wait_s(IB - 2, 2)
        wait_s(IB - 1, 3)

        if b + 1 < NBLK:
            pltpu.make_async_copy(src_hbm.at[w, b + 1], sidx.at[1 - q],
                                  isem).wait()
            pltpu.make_async_copy(dst_hbm.at[w, b + 1], didx.at[1 - q],
                                  isem).wait()

    plsc.subcore_barrier()

    for r in range(RPT // ZR):
        base = s * RPT + r * ZR
        pltpu.async_copy(acc.at[pl.ds(base, ZR)],
                         out_hbm.at[c].at[pl.ds(base, ZR)], sems[r % NBUF])
    for r in range(RPT // ZR):
        base = s * RPT + r * ZR
        pltpu.make_async_copy(acc.at[pl.ds(base, ZR)],
                              out_hbm.at[c].at[pl.ds(base, ZR)],
                              sems[r % NBUF]).wait()


@functools.cache
def _sc_call():
    return pl.kernel(
        _sc_body,
        out_type=jax.ShapeDtypeStruct((NC, AR, D), jnp.float32),
        mesh=plsc.VectorSubcoreMesh(core_axis_name="c", subcore_axis_name="s",
                                    num_cores=NC, num_subcores=NS),
        scratch_types=[
            pltpu.VMEM((2, IB, C), jnp.int32),
            pltpu.VMEM((2, IB, C), jnp.int32),
            pltpu.VMEM((C, D), jnp.float32),
            pltpu.VMEM((C, D), jnp.float32),
            pltpu.VMEM((C, D), jnp.float32),
            pltpu.VMEM((C, D), jnp.float32),
            pltpu.VMEM_SHARED((AR, D), jnp.float32),
            pltpu.SemaphoreType.DMA,
            pltpu.SemaphoreType.DMA,
            pltpu.SemaphoreType.DMA,
            pltpu.SemaphoreType.DMA,
            pltpu.SemaphoreType.DMA,
        ],
    )


def _combine_body(p_ref, o_ref):
    o_ref[...] = p_ref[0] + p_ref[1]


def kernel(i_node, edge_index):
    src = edge_index[0].astype(jnp.int32).reshape(NW, NBLK, IB, C)
    dst = edge_index[1].astype(jnp.int32).reshape(NW, NBLK, IB, C)
    partials = _sc_call()(src, dst, i_node)
    nb = 10
    rb = N_D // nb
    return pl.pallas_call(
        _combine_body,
        out_shape=jax.ShapeDtypeStruct((N_D, D), jnp.float32),
        grid=(nb,),
        in_specs=[pl.BlockSpec((NC, rb, D), lambda i: (0, i, 0))],
        out_specs=pl.BlockSpec((rb, D), lambda i: (i, 0)),
    )(partials)


# speedup vs baseline: 13.2881x; 1.0830x over previous
import functools

import jax
import jax.numpy as jnp
from jax import lax
from jax.experimental import pallas as pl
from jax.experimental.pallas import tpu as pltpu
from jax.experimental.pallas import tpu_sc as plsc

N_I = 10000
N_D = 10000
E = 320000
D = 128

NC = 2
NS = 16
NW = NC * NS
EPW = E // NW
C = 50
NBLK = 5
IB = 40
BE = IB * C
AR = 10240
RPT = AR // NS
ZB = 40
ZR = 128
NBUF = 4


def _sc_body(edge_hbm, table_hbm, out_hbm,
             sidx, didx, r0, r1, r2, r3, acc, s0, s1, s2, s3, isem):
    c = lax.axis_index("c")
    s = lax.axis_index("s")
    w = c * NS + s
    rows = (r0, r1, r2, r3)
    sems = (s0, s1, s2, s3)

    zero = jnp.zeros((16,), jnp.float32)

    @pl.loop(0, ZB)
    def _(i):
        @pl.loop(0, D // 16)
        def _(k):
            r0[i, pl.ds(k * 16, 16)] = zero

    zsrc = r0.at[pl.ds(0, ZB)]
    for r in range(RPT // ZB):
        pltpu.async_copy(zsrc, acc.at[pl.ds(s * RPT + r * ZB, ZB)], s0)
    for r in range(RPT // ZB):
        pltpu.make_async_copy(zsrc, acc.at[pl.ds(s * RPT, ZB)], s0).wait()
    plsc.subcore_barrier()

    src_all = edge_hbm.at[0]
    dst_all = edge_hbm.at[1]
    pltpu.sync_copy(src_all.at[w, 0], sidx.at[0])
    pltpu.sync_copy(dst_all.at[w, 0], didx.at[0])
    for b in range(NBLK):
        q = b % 2
        sq, dq = sidx.at[q], didx.at[q]

        def start_g(t, p, sq=sq):
            pltpu.async_copy(table_hbm.at[sq.at[t]], rows[p], sems[p])

        def wait_g(t, p, sq=sq):
            pltpu.make_async_copy(table_hbm.at[sq.at[t]], rows[p],
                                  sems[p]).wait()

        def start_s(t, p, dq=dq):
            pltpu.async_copy(rows[p], acc.at[dq.at[t]], sems[p], add=True)

        def wait_s(t, p, dq=dq):
            pltpu.make_async_copy(rows[p], acc.at[dq.at[t]], sems[p]).wait()

        if b + 1 < NBLK:
            pltpu.async_copy(src_all.at[w, b + 1], sidx.at[1 - q], isem)
            pltpu.async_copy(dst_all.at[w, b + 1], didx.at[1 - q], isem)

        start_g(0, 0)
        start_g(1, 1)
        wait_g(0, 0); start_s(0, 0); start_g(2, 2)
        wait_g(1, 1); start_s(1, 1); start_g(3, 3)
        wait_s(0, 0); start_g(4, 0); wait_g(2, 2); start_s(2, 2)
        wait_s(1, 1); start_g(5, 1); wait_g(3, 3); start_s(3, 3)

        @pl.loop(4, IB - 4, step=NBUF)
        def _(g):
            for p in range(NBUF):
                t = g + p
                qq = (p + 2) % NBUF
                wait_s(t - 2, qq)
                start_g(t + 2, qq)
                wait_g(t, p)
                start_s(t, p)

        wait_s(IB - 6, 2); start_g(IB - 2, 2); wait_g(IB - 4, 0); start_s(IB - 4, 0)
        wait_s(IB - 5, 3); start_g(IB - 1, 3); wait_g(IB - 3, 1); start_s(IB - 3, 1)
        wait_s(IB - 4, 0); wait_g(IB - 2, 2); start_s(IB - 2, 2)
        wait_s(IB - 3, 1); wait_g(IB - 1, 3); start_s(IB - 1, 3)
        wait_s(IB - 2, 2)
        wait_s(IB - 1, 3)

        if b + 1 < NBLK:
            pltpu.make_async_copy(src_all.at[w, b + 1], sidx.at[1 - q],
                                  isem).wait()
            pltpu.make_async_copy(dst_all.at[w, b + 1], didx.at[1 - q],
                                  isem).wait()

    plsc.subcore_barrier()

    for r in range(RPT // ZR):
        base = s * RPT + r * ZR
        pltpu.async_copy(acc.at[pl.ds(base, ZR)],
                         out_hbm.at[c].at[pl.ds(base, ZR)], sems[r % NBUF])
    for r in range(RPT // ZR):
        base = s * RPT + r * ZR
        pltpu.make_async_copy(acc.at[pl.ds(base, ZR)],
                              out_hbm.at[c].at[pl.ds(base, ZR)],
                              sems[r % NBUF]).wait()


@functools.cache
def _sc_call():
    return pl.kernel(
        _sc_body,
        out_type=jax.ShapeDtypeStruct((NC, AR, D), jnp.float32),
        mesh=plsc.VectorSubcoreMesh(core_axis_name="c", subcore_axis_name="s",
                                    num_cores=NC, num_subcores=NS),
        scratch_types=[
            pltpu.VMEM((2, IB, C), jnp.int32),
            pltpu.VMEM((2, IB, C), jnp.int32),
            pltpu.VMEM((C, D), jnp.float32),
            pltpu.VMEM((C, D), jnp.float32),
            pltpu.VMEM((C, D), jnp.float32),
            pltpu.VMEM((C, D), jnp.float32),
            pltpu.VMEM_SHARED((AR, D), jnp.float32),
            pltpu.SemaphoreType.DMA,
            pltpu.SemaphoreType.DMA,
            pltpu.SemaphoreType.DMA,
            pltpu.SemaphoreType.DMA,
            pltpu.SemaphoreType.DMA,
        ],
    )


def _combine_body(p_ref, o_ref):
    o_ref[...] = p_ref[0] + p_ref[1]


def kernel(i_node, edge_index):
    edges = edge_index.astype(jnp.int32).reshape(2, NW, NBLK, IB, C)
    partials = _sc_call()(edges, i_node)
    nb = 10
    rb = N_D // nb
    return pl.pallas_call(
        _combine_body,
        out_shape=jax.ShapeDtypeStruct((N_D, D), jnp.float32),
        grid=(nb,),
        in_specs=[pl.BlockSpec((NC, rb, D), lambda i: (0, i, 0))],
        out_specs=pl.BlockSpec((rb, D), lambda i: (i, 0)),
    )(partials)


# speedup vs baseline: 14.1050x; 1.0615x over previous
import functools

import jax
import jax.numpy as jnp
from jax import lax
from jax.experimental import pallas as pl
from jax.experimental.pallas import tpu as pltpu
from jax.experimental.pallas import tpu_sc as plsc

N_I = 10000
N_D = 10000
E = 320000
D = 128

NC = 2
NS = 16
NW = NC * NS
EPW = E // NW
C = 50
NBLK = 5
IB = 40
BE = IB * C
AR = 10240
RPT = AR // NS
ZB = 40
ZR = 128
NBUF = 4


def _sc_body(edge_hbm, table_hbm, out_hbm,
             sidx, didx, r0, r1, r2, r3, acc, s0, s1, s2, s3, isem):
    c = lax.axis_index("c")
    s = lax.axis_index("s")
    w = c * NS + s
    rows = (r0, r1, r2, r3)
    sems = (s0, s1, s2, s3)

    zero = jnp.zeros((16,), jnp.float32)

    @pl.loop(0, ZB)
    def _(i):
        @pl.loop(0, D // 16)
        def _(k):
            r0[i, pl.ds(k * 16, 16)] = zero

    zsrc = r0.at[pl.ds(0, ZB)]
    for r in range(RPT // ZB):
        pltpu.async_copy(zsrc, acc.at[pl.ds(s * RPT + r * ZB, ZB)], s0)
    for r in range(RPT // ZB):
        pltpu.make_async_copy(zsrc, acc.at[pl.ds(s * RPT, ZB)], s0).wait()
    plsc.subcore_barrier()

    src_all = edge_hbm.at[0]
    dst_all = edge_hbm.at[1]
    pltpu.sync_copy(src_all.at[w, 0], sidx.at[0])
    pltpu.sync_copy(dst_all.at[w, 0], didx.at[0])
    for b in range(NBLK):
        q = b % 2
        sq, dq = sidx.at[q], didx.at[q]

        def start_g(t, p, sq=sq):
            pltpu.async_copy(table_hbm.at[sq.at[t]], rows[p], sems[p])

        def wait_g(t, p, sq=sq):
            pltpu.make_async_copy(table_hbm.at[sq.at[t]], rows[p],
                                  sems[p]).wait()

        def start_s(t, p, dq=dq):
            pltpu.async_copy(rows[p], acc.at[dq.at[t]], sems[p], add=True)

        def wait_s(t, p, dq=dq):
            pltpu.make_async_copy(rows[p], acc.at[dq.at[t]], sems[p]).wait()

        if b + 1 < NBLK:
            pltpu.async_copy(src_all.at[w, b + 1], sidx.at[1 - q], isem)
            pltpu.async_copy(dst_all.at[w, b + 1], didx.at[1 - q], isem)

        start_g(0, 0); start_g(1, 1); start_g(2, 2)
        wait_g(0, 0); start_s(0, 0); start_g(3, 3)
        wait_s(0, 0); start_g(4, 0); wait_g(1, 1); start_s(1, 1)
        wait_s(1, 1); start_g(5, 1); wait_g(2, 2); start_s(2, 2)
        wait_s(2, 2); start_g(6, 2); wait_g(3, 3); start_s(3, 3)

        @pl.loop(4, IB - 4, step=NBUF)
        def _(g):
            for p in range(NBUF):
                t = g + p
                qq = (p + 3) % NBUF
                wait_s(t - 1, qq)
                start_g(t + 3, qq)
                wait_g(t, p)
                start_s(t, p)

        wait_s(IB - 5, 3); start_g(IB - 1, 3); wait_g(IB - 4, 0); start_s(IB - 4, 0)
        wait_s(IB - 4, 0); wait_g(IB - 3, 1); start_s(IB - 3, 1)
        wait_s(IB - 3, 1); wait_g(IB - 2, 2); start_s(IB - 2, 2)
        wait_s(IB - 2, 2); wait_g(IB - 1, 3); start_s(IB - 1, 3)
        wait_s(IB - 1, 3)

        if b + 1 < NBLK:
            pltpu.make_async_copy(src_all.at[w, b + 1], sidx.at[1 - q],
                                  isem).wait()
            pltpu.make_async_copy(dst_all.at[w, b + 1], didx.at[1 - q],
                                  isem).wait()

    plsc.subcore_barrier()

    for r in range(RPT // ZR):
        base = s * RPT + r * ZR
        pltpu.async_copy(acc.at[pl.ds(base, ZR)],
                         out_hbm.at[c].at[pl.ds(base, ZR)], sems[r % NBUF])
    for r in range(RPT // ZR):
        base = s * RPT + r * ZR
        pltpu.make_async_copy(acc.at[pl.ds(base, ZR)],
                              out_hbm.at[c].at[pl.ds(base, ZR)],
                              sems[r % NBUF]).wait()


@functools.cache
def _sc_call():
    return pl.kernel(
        _sc_body,
        out_type=jax.ShapeDtypeStruct((NC, AR, D), jnp.float32),
        mesh=plsc.VectorSubcoreMesh(core_axis_name="c", subcore_axis_name="s",
                                    num_cores=NC, num_subcores=NS),
        scratch_types=[
            pltpu.VMEM((2, IB, C), jnp.int32),
            pltpu.VMEM((2, IB, C), jnp.int32),
            pltpu.VMEM((C, D), jnp.float32),
            pltpu.VMEM((C, D), jnp.float32),
            pltpu.VMEM((C, D), jnp.float32),
            pltpu.VMEM((C, D), jnp.float32),
            pltpu.VMEM_SHARED((AR, D), jnp.float32),
            pltpu.SemaphoreType.DMA,
            pltpu.SemaphoreType.DMA,
            pltpu.SemaphoreType.DMA,
            pltpu.SemaphoreType.DMA,
            pltpu.SemaphoreType.DMA,
        ],
    )


def _combine_body(p_ref, o_ref):
    o_ref[...] = p_ref[0] + p_ref[1]


def kernel(i_node, edge_index):
    edges = edge_index.astype(jnp.int32).reshape(2, NW, NBLK, IB, C)
    partials = _sc_call()(edges, i_node)
    nb = 10
    rb = N_D // nb
    return pl.pallas_call(
        _combine_body,
        out_shape=jax.ShapeDtypeStruct((N_D, D), jnp.float32),
        grid=(nb,),
        in_specs=[pl.BlockSpec((NC, rb, D), lambda i: (0, i, 0))],
        out_specs=pl.BlockSpec((rb, D), lambda i: (i, 0)),
    )(partials)
